# baseline (speedup 1.0000x reference)
import functools

import jax
import jax.numpy as jnp
from jax import lax
from jax.experimental import pallas as pl
from jax.experimental.pallas import tpu as pltpu
from jax.experimental.pallas import tpu_sc as plsc

_L = 16
_NC = 2
_NS = 16
_NW = _NC * _NS
_G = 4


@functools.cache
def _build(n_events: int, n_nodes: int):
    assert n_events % (_NW * _L * _G) == 0
    ev_per_w = n_events // _NW
    n_groups = ev_per_w // _L
    mesh = plsc.VectorSubcoreMesh(core_axis_name="c", subcore_axis_name="s")

    @functools.partial(
        pl.kernel,
        out_type=[
            jax.ShapeDtypeStruct((_NW * _L,), jnp.float32),
            jax.ShapeDtypeStruct((_NW * _L,), jnp.float32),
        ],
        mesh=mesh,
        scratch_types=[
            pltpu.VMEM((ev_per_w * 3,), jnp.int32),
            pltpu.VMEM((n_nodes * 2,), jnp.float32),
            pltpu.VMEM((_L,), jnp.float32),
            pltpu.VMEM((_L,), jnp.float32),
        ],
        compiler_params=pltpu.CompilerParams(needs_layout_passes=False),
    )
    def sc_kernel(data_hbm, z_hbm, sd_out, se_out, d_v, z_v, oa_v, ob_v):
        wid = lax.axis_index("s") * _NC + lax.axis_index("c")
        base = wid * ev_per_w
        pltpu.sync_copy(z_hbm, z_v)
        pltpu.sync_copy(data_hbm.at[pl.ds(base * 3, ev_per_w * 3)], d_v)

        iota3 = lax.iota(jnp.int32, _L) * 3
        one = jnp.ones((_L,), jnp.int32)
        zero = jnp.zeros((_L,), jnp.float32)

        @plsc.parallel_loop(0, n_groups, step=_G, unroll=2,
                            carry=(zero,) * (2 * _G))
        def accs(t, carry):
            out = []
            for g in range(_G):
                rows3 = (t + g) * (_L * 3) + iota3
                iv = plsc.load_gather(d_v, [rows3])
                jv = plsc.load_gather(d_v, [rows3 + one])
                iv2 = iv + iv
                jv2 = jv + jv
                xi = plsc.load_gather(z_v, [iv2])
                yi = plsc.load_gather(z_v, [iv2 + one])
                xj = plsc.load_gather(z_v, [jv2])
                yj = plsc.load_gather(z_v, [jv2 + one])
                dx = xi - xj
                dy = yi - yj
                d = dx * dx + dy * dy
                out.append(carry[2 * g] + d)
                out.append(carry[2 * g + 1] + jnp.exp(-d))
            return tuple(out)

        acc_d = accs[0] + accs[2] + accs[4] + accs[6]
        acc_e = accs[1] + accs[3] + accs[5] + accs[7]
        oa_v[...] = acc_d
        ob_v[...] = acc_e
        pltpu.sync_copy(oa_v, sd_out.at[pl.ds(wid * _L, _L)])
        pltpu.sync_copy(ob_v, se_out.at[pl.ds(wid * _L, _L)])

    return sc_kernel


def kernel(data, t0, tn, beta, z0):
    n_events = data.shape[0]
    n_nodes = z0.shape[0]
    data_flat = jnp.reshape(data.astype(jnp.int32), (-1,))
    z_flat = jnp.reshape(z0.astype(jnp.float32), (-1,))
    sd_part, se_part = _build(n_events, n_nodes)(data_flat, z_flat)
    b = beta.astype(jnp.float32)[0, 0]
    event_intensity = n_events * b - jnp.sum(sd_part)
    non_event_intensity = jnp.exp(b) * jnp.sum(se_part)
    return -(event_intensity - non_event_intensity)


# speedup vs baseline: 5.4274x; 5.4274x over previous
import functools

import jax
import jax.numpy as jnp
from jax import lax
from jax.experimental import pallas as pl
from jax.experimental.pallas import tpu as pltpu
from jax.experimental.pallas import tpu_sc as plsc

_L = 16
_NC = 2
_NS = 16
_NW = _NC * _NS
_G = 4


@functools.cache
def _build(n_events: int, n_nodes: int):
    assert n_events % (_NW * _L * _G) == 0
    ev_per_w = n_events // _NW
    n_groups = ev_per_w // _L
    mesh = plsc.VectorSubcoreMesh(core_axis_name="c", subcore_axis_name="s")

    @functools.partial(
        pl.kernel,
        out_type=[
            jax.ShapeDtypeStruct((_NW * _L,), jnp.float32),
            jax.ShapeDtypeStruct((_NW * _L,), jnp.float32),
        ],
        mesh=mesh,
        scratch_types=[
            pltpu.VMEM((ev_per_w,), jnp.int32),
            pltpu.VMEM((ev_per_w,), jnp.int32),
            pltpu.VMEM((n_nodes,), jnp.float32),
            pltpu.VMEM((n_nodes,), jnp.float32),
            pltpu.VMEM((_L,), jnp.float32),
            pltpu.VMEM((_L,), jnp.float32),
        ],
        compiler_params=pltpu.CompilerParams(needs_layout_passes=False),
    )
    def sc_kernel(i_hbm, j_hbm, x_hbm, y_hbm, sd_out, se_out,
                  i_v, j_v, x_v, y_v, oa_v, ob_v):
        wid = lax.axis_index("s") * _NC + lax.axis_index("c")
        base = wid * ev_per_w
        pltpu.sync_copy(x_hbm, x_v)
        pltpu.sync_copy(y_hbm, y_v)
        pltpu.sync_copy(i_hbm.at[pl.ds(base, ev_per_w)], i_v)
        pltpu.sync_copy(j_hbm.at[pl.ds(base, ev_per_w)], j_v)

        zero = jnp.zeros((_L,), jnp.float32)

        @plsc.parallel_loop(0, n_groups, step=_G, unroll=2,
                            carry=(zero,) * (2 * _G))
        def accs(t, carry):
            out = []
            for g in range(_G):
                off = (t + g) * _L
                iv = i_v[pl.ds(off, _L)]
                jv = j_v[pl.ds(off, _L)]
                xi = plsc.load_gather(x_v, [iv])
                yi = plsc.load_gather(y_v, [iv])
                xj = plsc.load_gather(x_v, [jv])
                yj = plsc.load_gather(y_v, [jv])
                dx = xi - xj
                dy = yi - yj
                d = dx * dx + dy * dy
                out.append(carry[2 * g] + d)
                out.append(carry[2 * g + 1] + jnp.exp(-d))
            return tuple(out)

        acc_d = accs[0] + accs[2] + accs[4] + accs[6]
        acc_e = accs[1] + accs[3] + accs[5] + accs[7]
        oa_v[...] = acc_d
        ob_v[...] = acc_e
        pltpu.sync_copy(oa_v, sd_out.at[pl.ds(wid * _L, _L)])
        pltpu.sync_copy(ob_v, se_out.at[pl.ds(wid * _L, _L)])

    return sc_kernel


def kernel(data, t0, tn, beta, z0):
    n_events = data.shape[0]
    n_nodes = z0.shape[0]
    i_arr = data[:, 0].astype(jnp.int32)
    j_arr = data[:, 1].astype(jnp.int32)
    x_arr = z0[:, 0].astype(jnp.float32)
    y_arr = z0[:, 1].astype(jnp.float32)
    sd_part, se_part = _build(n_events, n_nodes)(i_arr, j_arr, x_arr, y_arr)
    b = beta.astype(jnp.float32)[0, 0]
    event_intensity = n_events * b - jnp.sum(sd_part)
    non_event_intensity = jnp.exp(b) * jnp.sum(se_part)
    return -(event_intensity - non_event_intensity)


# speedup vs baseline: 5.5566x; 1.0238x over previous
import functools

import jax
import jax.numpy as jnp
from jax import lax
from jax.experimental import pallas as pl
from jax.experimental.pallas import tpu as pltpu
from jax.experimental.pallas import tpu_sc as plsc

_L = 16
_NC = 2
_NS = 16
_NW = _NC * _NS
_G = 4


@functools.cache
def _build(n_events: int, n_nodes: int, shift: int):
    assert n_events % (_NW * _L * _G) == 0
    ev_per_w = n_events // _NW
    n_groups = ev_per_w // _L
    mesh = plsc.VectorSubcoreMesh(core_axis_name="c", subcore_axis_name="s")

    @functools.partial(
        pl.kernel,
        out_type=[
            jax.ShapeDtypeStruct((_NW * _L,), jnp.float32),
            jax.ShapeDtypeStruct((_NW * _L,), jnp.float32),
        ],
        mesh=mesh,
        scratch_types=[
            pltpu.VMEM((ev_per_w,), jnp.int32),
            pltpu.VMEM((n_nodes,), jnp.float32),
            pltpu.VMEM((n_nodes,), jnp.float32),
            pltpu.VMEM((_L,), jnp.float32),
            pltpu.VMEM((_L,), jnp.float32),
        ],
        compiler_params=pltpu.CompilerParams(needs_layout_passes=False),
    )
    def sc_kernel(ij_hbm, x_hbm, y_hbm, sd_out, se_out,
                  ij_v, x_v, y_v, oa_v, ob_v):
        wid = lax.axis_index("s") * _NC + lax.axis_index("c")
        base = wid * ev_per_w
        pltpu.sync_copy(x_hbm, x_v)
        pltpu.sync_copy(y_hbm, y_v)
        pltpu.sync_copy(ij_hbm.at[pl.ds(base, ev_per_w)], ij_v)

        zero = jnp.zeros((_L,), jnp.float32)
        mask = jnp.full((_L,), (1 << shift) - 1, jnp.int32)
        shift_v = jnp.full((_L,), shift, jnp.int32)

        @plsc.parallel_loop(0, n_groups, step=_G, unroll=2,
                            carry=(zero,) * (2 * _G))
        def accs(t, carry):
            out = []
            for g in range(_G):
                off = (t + g) * _L
                ij = ij_v[pl.ds(off, _L)]
                iv = lax.shift_right_logical(ij, shift_v)
                jv = lax.bitwise_and(ij, mask)
                xi = plsc.load_gather(x_v, [iv])
                yi = plsc.load_gather(y_v, [iv])
                xj = plsc.load_gather(x_v, [jv])
                yj = plsc.load_gather(y_v, [jv])
                dx = xi - xj
                dy = yi - yj
                d = dx * dx + dy * dy
                out.append(carry[2 * g] + d)
                out.append(carry[2 * g + 1] + jnp.exp(-d))
            return tuple(out)

        acc_d = accs[0] + accs[2] + accs[4] + accs[6]
        acc_e = accs[1] + accs[3] + accs[5] + accs[7]
        oa_v[...] = acc_d
        ob_v[...] = acc_e
        pltpu.sync_copy(oa_v, sd_out.at[pl.ds(wid * _L, _L)])
        pltpu.sync_copy(ob_v, se_out.at[pl.ds(wid * _L, _L)])

    return sc_kernel


def kernel(data, t0, tn, beta, z0):
    n_events = data.shape[0]
    n_nodes = z0.shape[0]
    shift = max(1, (n_nodes - 1).bit_length())
    ij_arr = jnp.left_shift(data[:, 0].astype(jnp.int32), shift) | \
        data[:, 1].astype(jnp.int32)
    x_arr = z0[:, 0].astype(jnp.float32)
    y_arr = z0[:, 1].astype(jnp.float32)
    sd_part, se_part = _build(n_events, n_nodes, shift)(ij_arr, x_arr, y_arr)
    b = beta.astype(jnp.float32)[0, 0]
    event_intensity = n_events * b - jnp.sum(sd_part)
    non_event_intensity = jnp.exp(b) * jnp.sum(se_part)
    return -(event_intensity - non_event_intensity)
